# Initial kernel scaffold; baseline (speedup 1.0000x reference)
import jax
import jax.numpy as jnp
from jax.experimental import pallas as pl


def kernel(feats, edge_index, W1, b1, W2, b2, Wc1, bc1, Wc2, bc2, Wc3, bc3):
    raise NotImplementedError("write your pallas kernel here")


# speedup vs baseline: 52.7148x; 52.7148x over previous
import functools

import jax
import jax.numpy as jnp
from jax import lax
from jax.experimental import pallas as pl
from jax.experimental.pallas import tpu as pltpu
from jax.experimental.pallas import tpu_sc as plsc

N_NODES = 100000
N_EDGES = 6400000
IN_DIM = 2
HIDDEN = 128

NC, NS = 2, 16
NW = NC * NS
E_PER_W = N_EDGES // NW
CE = 2000
NCHUNK = E_PER_W // CE
L = 16
NPT = 6256
NPT_LAST = N_NODES - (NS - 1) * NPT
APT = 12504
APT_LAST = 2 * N_NODES - (NS - 1) * APT


def _sc_body(src_hbm, dst_hbm, ft0_hbm, ft1_hbm, zeros_hbm, out_hbm,
             src_v, dst_v, d0_v, d1_v, v0, v1, fs0, fs1, accf, sem):
    cid = lax.axis_index("c")
    sid = lax.axis_index("s")
    wid = sid * NC + cid

    nbase = sid * NPT
    abase = sid * APT

    @pl.when(sid < NS - 1)
    def _():
        pltpu.sync_copy(ft0_hbm.at[pl.ds(nbase, NPT)], fs0.at[pl.ds(nbase, NPT)])
        pltpu.sync_copy(ft1_hbm.at[pl.ds(nbase, NPT)], fs1.at[pl.ds(nbase, NPT)])
        pltpu.sync_copy(zeros_hbm.at[pl.ds(abase, APT)], accf.at[pl.ds(abase, APT)])

    @pl.when(sid == NS - 1)
    def _():
        pltpu.sync_copy(ft0_hbm.at[pl.ds(nbase, NPT_LAST)], fs0.at[pl.ds(nbase, NPT_LAST)])
        pltpu.sync_copy(ft1_hbm.at[pl.ds(nbase, NPT_LAST)], fs1.at[pl.ds(nbase, NPT_LAST)])
        pltpu.sync_copy(zeros_hbm.at[pl.ds(abase, APT_LAST)], accf.at[pl.ds(abase, APT_LAST)])

    plsc.subcore_barrier()

    e_base = wid * E_PER_W

    def chunk(i, carry):
        e0 = e_base + i * CE
        pltpu.sync_copy(src_hbm.at[pl.ds(e0, CE)], src_v)
        pltpu.sync_copy(dst_hbm.at[pl.ds(e0, CE)], dst_v)

        def ixc(k, c):
            t = dst_v[pl.ds(k * L, L)]
            t2 = t + t
            d0_v[pl.ds(k * L, L)] = t2
            d1_v[pl.ds(k * L, L)] = t2 + 1
            return c

        lax.fori_loop(0, CE // L, ixc, 0)

        pltpu.async_copy(fs0.at[src_v], v0, sem).wait()
        pltpu.async_copy(fs1.at[src_v], v1, sem).wait()
        pltpu.sync_copy(v0, accf.at[d0_v], add=True)
        pltpu.sync_copy(v1, accf.at[d1_v], add=True)
        return carry

    lax.fori_loop(0, NCHUNK, chunk, 0)

    plsc.subcore_barrier()

    @pl.when(sid < NS - 1)
    def _():
        pltpu.sync_copy(accf.at[pl.ds(abase, APT)],
                        out_hbm.at[cid, pl.ds(abase, APT)])

    @pl.when(sid == NS - 1)
    def _():
        pltpu.sync_copy(accf.at[pl.ds(abase, APT_LAST)],
                        out_hbm.at[cid, pl.ds(abase, APT_LAST)])


_sc_mesh = plsc.VectorSubcoreMesh(core_axis_name="c", subcore_axis_name="s",
                                  num_cores=NC, num_subcores=NS)

_sc_seg_sum = functools.partial(
    pl.kernel,
    out_type=jax.ShapeDtypeStruct((NC, 2 * N_NODES), jnp.float32),
    mesh=_sc_mesh,
    scratch_types=[
        pltpu.VMEM((CE,), jnp.int32),
        pltpu.VMEM((CE,), jnp.int32),
        pltpu.VMEM((CE,), jnp.int32),
        pltpu.VMEM((CE,), jnp.int32),
        pltpu.VMEM((CE,), jnp.float32),
        pltpu.VMEM((CE,), jnp.float32),
        pltpu.VMEM_SHARED((N_NODES,), jnp.float32),
        pltpu.VMEM_SHARED((N_NODES,), jnp.float32),
        pltpu.VMEM_SHARED((2 * N_NODES,), jnp.float32),
        pltpu.SemaphoreType.DMA,
    ],
    compiler_params=pltpu.CompilerParams(use_tc_tiling_on_sc=False),
)(_sc_body)


BLK = 1000
GRID = N_NODES // BLK


def _tc_body(feats_ref, a0_ref, a1_ref, w1_ref, b1_ref, w2_ref, b2_ref,
             wc1_ref, bc1_ref, wc2_ref, bc2_ref, wc3_ref, bc3_ref,
             out_ref, mx_ref):
    i = pl.program_id(0)
    x = feats_ref[...] + a0_ref[...] + a1_ref[...]
    w1 = w1_ref[...]
    h = jnp.tanh(x[:, 0:1] * w1[0:1, :] + x[:, 1:2] * w1[1:2, :]
                 + b1_ref[...])
    h = jnp.tanh(jnp.dot(h, w2_ref[...],
                         preferred_element_type=jnp.float32) + b2_ref[...])
    m = jnp.max(h, axis=0, keepdims=True)

    @pl.when(i == 0)
    def _():
        mx_ref[...] = m

    @pl.when(i > 0)
    def _():
        mx_ref[...] = jnp.maximum(mx_ref[...], m)

    @pl.when(i == GRID - 1)
    def _():
        p = mx_ref[...]
        c = jnp.dot(p, wc1_ref[...],
                    preferred_element_type=jnp.float32) + bc1_ref[...]
        c = jnp.where(c > 0, c, jnp.exp(c) - 1.0)
        c = jnp.dot(c, wc2_ref[...],
                    preferred_element_type=jnp.float32) + bc2_ref[...]
        c = jnp.where(c > 0, c, jnp.exp(c) - 1.0)
        out_ref[...] = (jnp.sum(c * wc3_ref[...], axis=1, keepdims=True)
                        + bc3_ref[...])


def _tc_dense(feats, a0, a1, W1, b1, W2, b2, Wc1, bc1, Wc2, bc2, wc3t, bc3):
    cfull = lambda shape: pl.BlockSpec(shape, lambda i: (0, 0))
    return pl.pallas_call(
        _tc_body,
        grid=(GRID,),
        in_specs=[
            pl.BlockSpec((BLK, IN_DIM), lambda i: (i, 0)),
            pl.BlockSpec((BLK, IN_DIM), lambda i: (i, 0)),
            pl.BlockSpec((BLK, IN_DIM), lambda i: (i, 0)),
            cfull((IN_DIM, HIDDEN)),
            cfull((1, HIDDEN)),
            cfull((HIDDEN, HIDDEN)),
            cfull((1, HIDDEN)),
            cfull((HIDDEN, HIDDEN)),
            cfull((1, HIDDEN)),
            cfull((HIDDEN, 32)),
            cfull((1, 32)),
            cfull((1, 32)),
            cfull((1, 1)),
        ],
        out_specs=pl.BlockSpec((1, 1), lambda i: (0, 0)),
        out_shape=jax.ShapeDtypeStruct((1, 1), jnp.float32),
        scratch_shapes=[pltpu.VMEM((1, HIDDEN), jnp.float32)],
        compiler_params=pltpu.CompilerParams(
            dimension_semantics=("arbitrary",)),
    )(feats, a0, a1, W1, b1, W2, b2, Wc1, bc1, Wc2, bc2, wc3t, bc3)


def kernel(feats, edge_index, W1, b1, W2, b2, Wc1, bc1, Wc2, bc2, Wc3, bc3):
    ei = edge_index.astype(jnp.int32)
    ft = feats.T
    zeros = jnp.zeros((2 * N_NODES,), jnp.float32)
    aggf = _sc_seg_sum(ei[0], ei[1], ft[0], ft[1], zeros)
    agg2 = aggf.reshape(NC, N_NODES, IN_DIM)
    return _tc_dense(
        feats, agg2[0], agg2[1],
        W1, b1.reshape(1, HIDDEN), W2, b2.reshape(1, HIDDEN),
        Wc1, bc1.reshape(1, HIDDEN), Wc2, bc2.reshape(1, 32),
        Wc3.reshape(1, 32), bc3.reshape(1, 1))


# speedup vs baseline: 110.7865x; 2.1016x over previous
import functools

import jax
import jax.numpy as jnp
from jax import lax
from jax.experimental import pallas as pl
from jax.experimental.pallas import tpu as pltpu
from jax.experimental.pallas import tpu_sc as plsc

N_NODES = 100000
N_EDGES = 6400000
IN_DIM = 2
HIDDEN = 128

NC, NS = 2, 16
NW = NC * NS
E_PER_W = N_EDGES // NW
CE = 8000
NCHUNK = E_PER_W // CE
NPT = 6256
NPT_LAST = N_NODES - (NS - 1) * NPT


def _sc_body(src_hbm, dst_hbm, ft0_hbm, ft1_hbm, zeros_hbm, out_hbm,
             src_v, dst_v, v0, v1, fs0, fs1, acc0, acc1, sem):
    cid = lax.axis_index("c")
    sid = lax.axis_index("s")
    wid = sid * NC + cid

    nbase = sid * NPT

    def stage(n):
        pltpu.sync_copy(ft0_hbm.at[pl.ds(nbase, n)], fs0.at[pl.ds(nbase, n)])
        pltpu.sync_copy(ft1_hbm.at[pl.ds(nbase, n)], fs1.at[pl.ds(nbase, n)])
        pltpu.sync_copy(zeros_hbm.at[pl.ds(nbase, n)], acc0.at[pl.ds(nbase, n)])
        pltpu.sync_copy(zeros_hbm.at[pl.ds(nbase, n)], acc1.at[pl.ds(nbase, n)])

    @pl.when(sid < NS - 1)
    def _():
        stage(NPT)

    @pl.when(sid == NS - 1)
    def _():
        stage(NPT_LAST)

    plsc.subcore_barrier()

    e_base = wid * E_PER_W

    def chunk(i, carry):
        e0 = e_base + i * CE
        pltpu.sync_copy(src_hbm.at[pl.ds(e0, CE)], src_v)
        pltpu.sync_copy(dst_hbm.at[pl.ds(e0, CE)], dst_v)
        pltpu.async_copy(fs0.at[src_v], v0, sem).wait()
        pltpu.async_copy(fs1.at[src_v], v1, sem).wait()
        pltpu.sync_copy(v0, acc0.at[dst_v], add=True)
        pltpu.sync_copy(v1, acc1.at[dst_v], add=True)
        return carry

    lax.fori_loop(0, NCHUNK, chunk, 0)

    plsc.subcore_barrier()

    def unstage(n):
        pltpu.sync_copy(acc0.at[pl.ds(nbase, n)],
                        out_hbm.at[cid, 0, pl.ds(nbase, n)])
        pltpu.sync_copy(acc1.at[pl.ds(nbase, n)],
                        out_hbm.at[cid, 1, pl.ds(nbase, n)])

    @pl.when(sid < NS - 1)
    def _():
        unstage(NPT)

    @pl.when(sid == NS - 1)
    def _():
        unstage(NPT_LAST)


_sc_mesh = plsc.VectorSubcoreMesh(core_axis_name="c", subcore_axis_name="s",
                                  num_cores=NC, num_subcores=NS)

_sc_seg_sum = functools.partial(
    pl.kernel,
    out_type=jax.ShapeDtypeStruct((NC, IN_DIM, N_NODES), jnp.float32),
    mesh=_sc_mesh,
    scratch_types=[
        pltpu.VMEM((CE,), jnp.int32),
        pltpu.VMEM((CE,), jnp.int32),
        pltpu.VMEM((CE,), jnp.float32),
        pltpu.VMEM((CE,), jnp.float32),
        pltpu.VMEM_SHARED((N_NODES,), jnp.float32),
        pltpu.VMEM_SHARED((N_NODES,), jnp.float32),
        pltpu.VMEM_SHARED((N_NODES,), jnp.float32),
        pltpu.VMEM_SHARED((N_NODES,), jnp.float32),
        pltpu.SemaphoreType.DMA,
    ],
    compiler_params=pltpu.CompilerParams(use_tc_tiling_on_sc=False),
)(_sc_body)


N_PAD = 100352
BLK = 2048
GRID = N_PAD // BLK


def _tc_body(ft_ref, agg_ref, w1t_ref, b1c_ref, w2t_ref, b2c_ref,
             wc1t_ref, bc1c_ref, wc2t_ref, bc2c_ref, wc3c_ref, bc3_ref,
             out_ref, mx_ref):
    i = pl.program_id(0)
    a = agg_ref[...]
    x0 = ft_ref[0:1, :] + a[0:1, :] + a[2:3, :]
    x1 = ft_ref[1:2, :] + a[1:2, :] + a[3:4, :]
    w1t = w1t_ref[...]
    h = jnp.tanh(w1t[:, 0:1] * x0 + w1t[:, 1:2] * x1
                 + b1c_ref[...])
    h = jnp.tanh(jnp.dot(w2t_ref[...], h,
                         preferred_element_type=jnp.float32) + b2c_ref[...])
    lane = jax.lax.broadcasted_iota(jnp.int32, (HIDDEN, BLK), 1) + i * BLK
    h = jnp.where(lane < N_NODES, h, -2.0)
    m = jnp.max(h, axis=1, keepdims=True)

    @pl.when(i == 0)
    def _():
        mx_ref[...] = m

    @pl.when(i > 0)
    def _():
        mx_ref[...] = jnp.maximum(mx_ref[...], m)

    @pl.when(i == GRID - 1)
    def _():
        p = mx_ref[...]
        c = jnp.dot(wc1t_ref[...], p,
                    preferred_element_type=jnp.float32) + bc1c_ref[...]
        c = jnp.where(c > 0, c, jnp.exp(c) - 1.0)
        c = jnp.dot(wc2t_ref[...], c,
                    preferred_element_type=jnp.float32) + bc2c_ref[...]
        c = jnp.where(c > 0, c, jnp.exp(c) - 1.0)
        out_ref[...] = (jnp.sum(c * wc3c_ref[...], axis=0, keepdims=True)
                        + bc3_ref[...])


def _tc_dense(ft, aggc, w1t, b1c, w2t, b2c, wc1t, bc1c, wc2t, bc2c, wc3c, bc3):
    cfull = lambda shape: pl.BlockSpec(shape, lambda i: (0, 0))
    return pl.pallas_call(
        _tc_body,
        grid=(GRID,),
        in_specs=[
            pl.BlockSpec((IN_DIM, BLK), lambda i: (0, i)),
            pl.BlockSpec((2 * IN_DIM, BLK), lambda i: (0, i)),
            cfull((HIDDEN, IN_DIM)),
            cfull((HIDDEN, 1)),
            cfull((HIDDEN, HIDDEN)),
            cfull((HIDDEN, 1)),
            cfull((HIDDEN, HIDDEN)),
            cfull((HIDDEN, 1)),
            cfull((32, HIDDEN)),
            cfull((32, 1)),
            cfull((32, 1)),
            cfull((1, 1)),
        ],
        out_specs=pl.BlockSpec((1, 1), lambda i: (0, 0)),
        out_shape=jax.ShapeDtypeStruct((1, 1), jnp.float32),
        scratch_shapes=[pltpu.VMEM((HIDDEN, 1), jnp.float32)],
        compiler_params=pltpu.CompilerParams(
            dimension_semantics=("arbitrary",)),
    )(ft, aggc, w1t, b1c, w2t, b2c, wc1t, bc1c, wc2t, bc2c, wc3c, bc3)


def kernel(feats, edge_index, W1, b1, W2, b2, Wc1, bc1, Wc2, bc2, Wc3, bc3):
    ei = edge_index.astype(jnp.int32)
    ft = feats.T
    zeros = jnp.zeros((N_NODES,), jnp.float32)
    aggp = _sc_seg_sum(ei[0], ei[1], ft[0], ft[1], zeros)
    aggc = aggp.reshape(NC * IN_DIM, N_NODES)
    pad = ((0, 0), (0, N_PAD - N_NODES))
    return _tc_dense(
        jnp.pad(ft, pad), jnp.pad(aggc, pad),
        W1.T, b1.reshape(HIDDEN, 1), W2.T, b2.reshape(HIDDEN, 1),
        Wc1.T, bc1.reshape(HIDDEN, 1), Wc2.T, bc2.reshape(32, 1),
        Wc3, bc3.reshape(1, 1))


# speedup vs baseline: 130.9911x; 1.1824x over previous
import functools

import jax
import jax.numpy as jnp
from jax import lax
from jax.experimental import pallas as pl
from jax.experimental.pallas import tpu as pltpu
from jax.experimental.pallas import tpu_sc as plsc

N_NODES = 100000
N_EDGES = 6400000
IN_DIM = 2
HIDDEN = 128

NC, NS = 2, 16
NW = NC * NS
E_PER_W = N_EDGES // NW
CE = 5000
NCHUNK = E_PER_W // CE
NRING = 4
NPT = 6256
NPT_LAST = N_NODES - (NS - 1) * NPT


def _sc_body(ei_hbm, ft0_hbm, ft1_hbm, zeros_hbm, out_hbm,
             sv0, sv1, sv2, sv3, dv0, dv1, dv2, dv3,
             v00, v01, v02, v03, v10, v11, v12, v13,
             fs0, fs1, acc0, acc1,
             semg, si0, si1, si2, si3, ss0, ss1, ss2, ss3):
    SV = (sv0, sv1, sv2, sv3)
    DV = (dv0, dv1, dv2, dv3)
    V0 = (v00, v01, v02, v03)
    V1 = (v10, v11, v12, v13)
    SI = (si0, si1, si2, si3)
    SS = (ss0, ss1, ss2, ss3)
    cid = lax.axis_index("c")
    sid = lax.axis_index("s")
    wid = sid * NC + cid

    nbase = sid * NPT

    def stage(n):
        pltpu.sync_copy(ft0_hbm.at[pl.ds(nbase, n)], fs0.at[pl.ds(nbase, n)])
        pltpu.sync_copy(ft1_hbm.at[pl.ds(nbase, n)], fs1.at[pl.ds(nbase, n)])
        pltpu.sync_copy(zeros_hbm.at[pl.ds(nbase, n)], acc0.at[pl.ds(nbase, n)])
        pltpu.sync_copy(zeros_hbm.at[pl.ds(nbase, n)], acc1.at[pl.ds(nbase, n)])

    @pl.when(sid < NS - 1)
    def _():
        stage(NPT)

    @pl.when(sid == NS - 1)
    def _():
        stage(NPT_LAST)

    plsc.subcore_barrier()

    e_base = wid * E_PER_W

    pltpu.async_copy(ei_hbm.at[0, pl.ds(e_base, CE)], SV[0], SI[0])
    pltpu.async_copy(ei_hbm.at[1, pl.ds(e_base, CE)], DV[0], SI[0])

    def pipeline(j, carry):
        for b in range(NRING):
            i = NRING * j + b
            d = (b + 1) % NRING

            def drain_scatters():
                pltpu.make_async_copy(ft0_hbm.at[pl.ds(0, CE)], V0[d], SS[d]).wait()
                pltpu.make_async_copy(ft0_hbm.at[pl.ds(0, CE)], V1[d], SS[d]).wait()

            if b == NRING - 1:
                drain_scatters()
            else:
                pl.when(j > 0)(drain_scatters)

            def prefetch_idx():
                e1 = e_base + (i + 1) * CE
                pltpu.async_copy(ei_hbm.at[0, pl.ds(e1, CE)], SV[d], SI[d])
                pltpu.async_copy(ei_hbm.at[1, pl.ds(e1, CE)], DV[d], SI[d])

            if b < NRING - 1:
                prefetch_idx()
            else:
                pl.when(j < NCHUNK // NRING - 1)(prefetch_idx)

            pltpu.make_async_copy(ei_hbm.at[0, pl.ds(0, CE)], SV[b], SI[b]).wait()
            pltpu.make_async_copy(ei_hbm.at[0, pl.ds(0, CE)], DV[b], SI[b]).wait()

            g0 = pltpu.async_copy(fs0.at[SV[b]], V0[b], semg)
            g1 = pltpu.async_copy(fs1.at[SV[b]], V1[b], semg)
            g0.wait()
            g1.wait()

            pltpu.async_copy(V0[b], acc0.at[DV[b]], SS[b], add=True)
            pltpu.async_copy(V1[b], acc1.at[DV[b]], SS[b], add=True)
        return carry

    lax.fori_loop(0, NCHUNK // NRING, pipeline, 0)

    for d in range(1, NRING):
        pltpu.make_async_copy(ft0_hbm.at[pl.ds(0, CE)], V0[d], SS[d]).wait()
        pltpu.make_async_copy(ft0_hbm.at[pl.ds(0, CE)], V1[d], SS[d]).wait()

    plsc.subcore_barrier()

    def unstage(n):
        pltpu.sync_copy(acc0.at[pl.ds(nbase, n)],
                        out_hbm.at[cid, 0, pl.ds(nbase, n)])
        pltpu.sync_copy(acc1.at[pl.ds(nbase, n)],
                        out_hbm.at[cid, 1, pl.ds(nbase, n)])

    @pl.when(sid < NS - 1)
    def _():
        unstage(NPT)

    @pl.when(sid == NS - 1)
    def _():
        unstage(NPT_LAST)


_sc_mesh = plsc.VectorSubcoreMesh(core_axis_name="c", subcore_axis_name="s",
                                  num_cores=NC, num_subcores=NS)

_sc_seg_sum = functools.partial(
    pl.kernel,
    out_type=jax.ShapeDtypeStruct((NC, IN_DIM, N_NODES), jnp.float32),
    mesh=_sc_mesh,
    scratch_types=(
        [pltpu.VMEM((CE,), jnp.int32) for _ in range(8)]
        + [pltpu.VMEM((CE,), jnp.float32) for _ in range(8)]
        + [pltpu.VMEM_SHARED((N_NODES,), jnp.float32) for _ in range(4)]
        + [pltpu.SemaphoreType.DMA for _ in range(9)]
    ),
    compiler_params=pltpu.CompilerParams(use_tc_tiling_on_sc=False),
)(_sc_body)


N_PAD = 100352
BLK = 2048
GRID = N_PAD // BLK


def _tc_body(ft_ref, agg_ref, w1t_ref, b1c_ref, w2t_ref, b2c_ref,
             wc1t_ref, bc1c_ref, wc2t_ref, bc2c_ref, wc3c_ref, bc3_ref,
             out_ref, mx_ref):
    i = pl.program_id(0)
    a = agg_ref[...]
    x0 = ft_ref[0:1, :] + a[0:1, :] + a[2:3, :]
    x1 = ft_ref[1:2, :] + a[1:2, :] + a[3:4, :]
    w1t = w1t_ref[...]
    h = jnp.tanh(w1t[:, 0:1] * x0 + w1t[:, 1:2] * x1
                 + b1c_ref[...])
    h = jnp.tanh(jnp.dot(w2t_ref[...], h,
                         preferred_element_type=jnp.float32) + b2c_ref[...])
    lane = jax.lax.broadcasted_iota(jnp.int32, (HIDDEN, BLK), 1) + i * BLK
    h = jnp.where(lane < N_NODES, h, -2.0)
    m = jnp.max(h, axis=1, keepdims=True)

    @pl.when(i == 0)
    def _():
        mx_ref[...] = m

    @pl.when(i > 0)
    def _():
        mx_ref[...] = jnp.maximum(mx_ref[...], m)

    @pl.when(i == GRID - 1)
    def _():
        p = mx_ref[...]
        c = jnp.dot(wc1t_ref[...], p,
                    preferred_element_type=jnp.float32) + bc1c_ref[...]
        c = jnp.where(c > 0, c, jnp.exp(c) - 1.0)
        c = jnp.dot(wc2t_ref[...], c,
                    preferred_element_type=jnp.float32) + bc2c_ref[...]
        c = jnp.where(c > 0, c, jnp.exp(c) - 1.0)
        out_ref[...] = (jnp.sum(c * wc3c_ref[...], axis=0, keepdims=True)
                        + bc3_ref[...])


def _tc_dense(ft, aggc, w1t, b1c, w2t, b2c, wc1t, bc1c, wc2t, bc2c, wc3c, bc3):
    cfull = lambda shape: pl.BlockSpec(shape, lambda i: (0, 0))
    return pl.pallas_call(
        _tc_body,
        grid=(GRID,),
        in_specs=[
            pl.BlockSpec((IN_DIM, BLK), lambda i: (0, i)),
            pl.BlockSpec((2 * IN_DIM, BLK), lambda i: (0, i)),
            cfull((HIDDEN, IN_DIM)),
            cfull((HIDDEN, 1)),
            cfull((HIDDEN, HIDDEN)),
            cfull((HIDDEN, 1)),
            cfull((HIDDEN, HIDDEN)),
            cfull((HIDDEN, 1)),
            cfull((32, HIDDEN)),
            cfull((32, 1)),
            cfull((32, 1)),
            cfull((1, 1)),
        ],
        out_specs=pl.BlockSpec((1, 1), lambda i: (0, 0)),
        out_shape=jax.ShapeDtypeStruct((1, 1), jnp.float32),
        scratch_shapes=[pltpu.VMEM((HIDDEN, 1), jnp.float32)],
        compiler_params=pltpu.CompilerParams(
            dimension_semantics=("arbitrary",)),
    )(ft, aggc, w1t, b1c, w2t, b2c, wc1t, bc1c, wc2t, bc2c, wc3c, bc3)


def kernel(feats, edge_index, W1, b1, W2, b2, Wc1, bc1, Wc2, bc2, Wc3, bc3):
    ei = edge_index.astype(jnp.int32)
    ft = feats.T
    zeros = jnp.zeros((N_NODES,), jnp.float32)
    aggp = _sc_seg_sum(ei, ft[0], ft[1], zeros)
    aggc = aggp.reshape(NC * IN_DIM, N_NODES)
    pad = ((0, 0), (0, N_PAD - N_NODES))
    return _tc_dense(
        jnp.pad(ft, pad), jnp.pad(aggc, pad),
        W1.T, b1.reshape(HIDDEN, 1), W2.T, b2.reshape(HIDDEN, 1),
        Wc1.T, bc1.reshape(HIDDEN, 1), Wc2.T, bc2.reshape(32, 1),
        Wc3, bc3.reshape(1, 1))
